# baseline (speedup 1.0000x reference)
import functools

import jax
import jax.numpy as jnp
import numpy as np
from jax import lax
from jax.experimental import pallas as pl
from jax.experimental.pallas import tpu as pltpu
from jax.experimental.pallas import tpu_sc as plsc

N_NODES = 10000
N_EDGES = 320000
D = 128
D_EDGE = 16

NC = 2
NS = 16
NW = NC * NS

CHUNK = 80
R = CHUNK // 2
E_PAD = 322560
E_HALF = E_PAD // 2
EPW = E_PAD // NW
TOT_CHUNKS = E_PAD // CHUNK
N0 = 146
N1 = 106
IDXQ = 6
N_AGG = 10112
ROWS_PER_TILE = N_AGG // NS


def _permute_cols(w):
    i = w.shape[0]
    return w.reshape(i, D // 32, 2, 16).transpose(0, 2, 1, 3).reshape(i, D)


def _permute_bias(b):
    return b.reshape(D // 32, 2, 16).transpose(1, 0, 2).reshape(D)


def _pack_bf16(v):
    lo = jax.lax.shift_right_logical(
        jax.lax.bitcast_convert_type(v[:, :D // 2], jnp.int32) + 0x8000, 16)
    hi = (jax.lax.bitcast_convert_type(v[:, D // 2:], jnp.int32)
          + 0x8000) & (-65536)
    return lo | hi


def _mm2_body(x_ref, w_ref, b_ref, wq_ref, bq_ref, o_ref, oq_ref):
    x = x_ref[...]
    o_ref[...] = (
        jnp.dot(x, w_ref[...], preferred_element_type=jnp.float32)
        + b_ref[...]
    )
    oq_ref[...] = _pack_bf16(
        jnp.dot(x, wq_ref[...], preferred_element_type=jnp.float32)
        + bq_ref[...]
    )


def _mm2(x, w, b, wq, bq, block_rows=2000):
    m, k = x.shape
    n = w.shape[1]
    grid = (m // block_rows,)
    return pl.pallas_call(
        _mm2_body,
        grid=grid,
        in_specs=[
            pl.BlockSpec((block_rows, k), lambda i: (i, 0)),
            pl.BlockSpec((k, n), lambda i: (0, 0)),
            pl.BlockSpec((1, n), lambda i: (0, 0)),
            pl.BlockSpec((k, n), lambda i: (0, 0)),
            pl.BlockSpec((1, n), lambda i: (0, 0)),
        ],
        out_specs=[
            pl.BlockSpec((block_rows, n), lambda i: (i, 0)),
            pl.BlockSpec((block_rows, n // 2), lambda i: (i, 0)),
        ],
        out_shape=[
            jax.ShapeDtypeStruct((m, n), jnp.float32),
            jax.ShapeDtypeStruct((m, n // 2), jnp.int32),
        ],
    )(x, w, b.reshape(1, n), wq, bq.reshape(1, n))


def _emm_body(xa_ref, xb_ref, w_ref, b_ref, o_ref):
    w_, b_ = w_ref[...], b_ref[...]
    o_ref[:, :D // 2] = _pack_bf16(
        jnp.dot(xa_ref[...], w_, preferred_element_type=jnp.float32) + b_)
    o_ref[:, D // 2:] = _pack_bf16(
        jnp.dot(xb_ref[...], w_, preferred_element_type=jnp.float32) + b_)


def _emm(x, w, b, block_rows=1280):
    m, k = x.shape
    n = w.shape[1]
    half_blocks = (m // 2) // block_rows
    return pl.pallas_call(
        _emm_body,
        grid=(half_blocks,),
        in_specs=[
            pl.BlockSpec((block_rows, k), lambda i: (i, 0)),
            pl.BlockSpec((block_rows, k),
                         lambda i, hb=half_blocks: (i + hb, 0)),
            pl.BlockSpec((k, n), lambda i: (0, 0)),
            pl.BlockSpec((1, n), lambda i: (0, 0)),
        ],
        out_specs=pl.BlockSpec((block_rows, n), lambda i: (i, 0)),
        out_shape=jax.ShapeDtypeStruct((m // 2, n), jnp.int32),
    )(x, x, w, b.reshape(1, n))


def _combine_mm_body(h_ref, a0_ref, a1_ref, skip_ref, wi_ref, wn_ref, bn_ref,
                     wnq_ref, bnq_ref, skip_o_ref, h_o_ref, hq_o_ref,
                     *, mm_from_skip):
    xl = jnp.maximum(h_ref[...] + a0_ref[...] + a1_ref[...], 0.0)
    skip_new = skip_ref[...] + wi_ref[0, 0] * xl
    skip_o_ref[...] = skip_new
    mm_in = skip_new if mm_from_skip else xl
    h_o_ref[...] = (
        jnp.dot(mm_in, wn_ref[...], preferred_element_type=jnp.float32)
        + bn_ref[...]
    )
    hq_o_ref[...] = _pack_bf16(
        jnp.dot(mm_in, wnq_ref[...], preferred_element_type=jnp.float32)
        + bnq_ref[...]
    )


def _combine_mm(h, a0, a1, skip, wi, wn, bn, wnq, bnq, mm_from_skip,
                block_rows=2000):
    m, n = h.shape
    grid = (m // block_rows,)
    body = functools.partial(_combine_mm_body, mm_from_skip=mm_from_skip)
    bspec = pl.BlockSpec((block_rows, n), lambda i: (i, 0))
    wspec = pl.BlockSpec((n, n), lambda i: (0, 0))
    b1spec = pl.BlockSpec((1, n), lambda i: (0, 0))
    return pl.pallas_call(
        body,
        grid=grid,
        in_specs=[bspec, bspec, bspec, bspec,
                  pl.BlockSpec(memory_space=pltpu.SMEM),
                  wspec, b1spec, wspec, b1spec],
        out_specs=[bspec, bspec,
                   pl.BlockSpec((block_rows, n // 2), lambda i: (i, 0))],
        out_shape=[
            jax.ShapeDtypeStruct((m, n), jnp.float32),
            jax.ShapeDtypeStruct((m, n), jnp.float32),
            jax.ShapeDtypeStruct((m, n // 2), jnp.int32),
        ],
    )(h, a0, a1, skip, wi.reshape(1, 1), wn, bn.reshape(1, n),
      wnq, bnq.reshape(1, n))


def _final_body(h_ref, a0_ref, a1_ref, o_ref):
    o_ref[...] = jnp.maximum(h_ref[...] + a0_ref[...] + a1_ref[...], 0.0)


def _final_combine(h, a0, a1, block_rows=2000):
    m, n = h.shape
    grid = (m // block_rows,)
    spec = pl.BlockSpec((block_rows, n), lambda i: (i, 0))
    return pl.pallas_call(
        _final_body,
        grid=grid,
        in_specs=[spec, spec, spec],
        out_specs=spec,
        out_shape=jax.ShapeDtypeStruct((m, n), jnp.float32),
    )(h, a0, a1)


def _sc_edge_body(h_hbm, e_hbm, src_hbm, dst_hbm, out_hbm,
                  src_v, dst_v, g_v, e_v, m_v, agg_sh,
                  sem_i, sem_g, sem_e, sem_s):
    cid = lax.axis_index("c")
    sid = lax.axis_index("s")
    base = jnp.where(cid == 0, sid * N0, NS * N0 + sid * N1)
    nck = jnp.where(cid == 0, N0, N1)

    def issue_idx(c):
        q = c % IDXQ
        pltpu.async_copy(src_hbm.at[base + c, :], src_v.at[q], sem_i)
        pltpu.async_copy(dst_hbm.at[base + c, :], dst_v.at[q], sem_i)

    def wait_idx(c):
        q = c % IDXQ
        pltpu.make_async_copy(src_hbm.at[base + c, :], src_v.at[q],
                              sem_i).wait()
        pltpu.make_async_copy(dst_hbm.at[base + c, :], dst_v.at[q],
                              sem_i).wait()

    def issue(c, b):
        pltpu.async_copy(h_hbm.at[src_v.at[c % IDXQ]], g_v[b], sem_g[b])
        pltpu.async_copy(e_hbm.at[pl.ds((base + c) * R, R), :],
                         e_v[b], sem_e[b])

    def wait_in(c, b):
        pltpu.make_async_copy(h_hbm.at[src_v.at[c % IDXQ]], g_v[b],
                              sem_g[b]).wait()
        pltpu.make_async_copy(e_hbm.at[pl.ds((base + c) * R, R), :],
                              e_v[b], sem_e[b]).wait()

    def compute(b):
        @plsc.parallel_loop(0, R, 1, unroll=8)
        def row_body(r):
            for half in range(2):
                mr = r if half == 0 else R + r
                for j in range(D // 32):
                    gi = g_v[b][mr, pl.ds(j * 16, 16)]
                    ei = e_v[b][r, pl.ds(half * 64 + j * 16, 16)]
                    glo = lax.bitcast_convert_type(gi << 16, jnp.float32)
                    elo = lax.bitcast_convert_type(ei << 16, jnp.float32)
                    ghi = lax.bitcast_convert_type(gi & (-65536), jnp.float32)
                    ehi = lax.bitcast_convert_type(ei & (-65536), jnp.float32)
                    m_v[b][mr, pl.ds(j * 32, 16)] = jnp.maximum(glo + elo, 0.0)
                    m_v[b][mr, pl.ds(j * 32 + 16, 16)] = (
                        jnp.maximum(ghi + ehi, 0.0))

    def issue_scatter(c, b):
        pltpu.async_copy(m_v[b], agg_sh.at[dst_v.at[c % IDXQ]],
                         sem_s[b], add=True)

    def wait_scatter(c, b):
        pltpu.make_async_copy(m_v[b], agg_sh.at[dst_v.at[c % IDXQ]],
                              sem_s[b]).wait()

    for c in range(4):
        issue_idx(c)

    def zero_row(r, _):
        for j in range(D // 16):
            m_v[0][r, pl.ds(j * 16, 16)] = jnp.zeros((16,), jnp.float32)
        return 0
    lax.fori_loop(0, CHUNK, zero_row, 0)
    row0 = sid * ROWS_PER_TILE
    for k in range((ROWS_PER_TILE + CHUNK - 1) // CHUNK):
        rows = min(CHUNK, ROWS_PER_TILE - k * CHUNK)
        pltpu.sync_copy(m_v[0].at[pl.ds(0, rows), :],
                        agg_sh.at[pl.ds(row0 + k * CHUNK, rows), :])
    plsc.subcore_barrier()

    wait_idx(0)
    wait_idx(1)
    for b in range(2):
        issue(b, b)
    for c in range(2):
        b = c
        wait_in(c, b)
        compute(b)
        wait_idx(c + 2)
        issue(c + 2, b)
        issue_idx(c + 4)
        issue_scatter(c, b)

    def pair_body(i, _):
        for b in range(2):
            c = 2 * i + b
            wait_in(c, b)
            wait_scatter(c - 2, b)
            compute(b)
            wait_idx(c + 2)
            issue(c + 2, b)
            issue_idx(c + 4)
            issue_scatter(c, b)
        return 0
    lax.fori_loop(1, nck // 2 - 2, pair_body, 0)

    for k in range(4):
        c = nck - 4 + k
        b = k % 2
        wait_in(c, b)
        wait_scatter(c - 2, b)
        compute(b)
        if k < 2:
            wait_idx(c + 2)
            issue(c + 2, b)
        issue_scatter(c, b)
    for b in range(2):
        wait_scatter(nck - 2 + b, b)
    plsc.subcore_barrier()

    for k in range((ROWS_PER_TILE + CHUNK - 1) // CHUNK):
        rows = min(CHUNK, ROWS_PER_TILE - k * CHUNK)
        pltpu.sync_copy(agg_sh.at[pl.ds(row0 + k * CHUNK, rows), :],
                        m_v[0].at[pl.ds(0, rows), :])
        pltpu.sync_copy(m_v[0].at[pl.ds(0, rows), :],
                        out_hbm.at[cid, pl.ds(row0 + k * CHUNK, rows), :])


@functools.lru_cache(maxsize=None)
def _sc_edge_kernel():
    return pl.kernel(
        _sc_edge_body,
        out_type=jax.ShapeDtypeStruct((NC, N_AGG, D), jnp.float32),
        mesh=plsc.VectorSubcoreMesh(core_axis_name="c", subcore_axis_name="s",
                                    num_cores=NC, num_subcores=NS),
        compiler_params=pltpu.CompilerParams(use_tc_tiling_on_sc=False),
        scratch_types=[
            pltpu.VMEM((IDXQ, CHUNK), jnp.int32),
            pltpu.VMEM((IDXQ, CHUNK), jnp.int32),
            [pltpu.VMEM((CHUNK, D // 2), jnp.int32) for _ in range(2)],
            [pltpu.VMEM((R, D), jnp.int32) for _ in range(2)],
            [pltpu.VMEM((CHUNK, D), jnp.float32) for _ in range(2)],
            pltpu.VMEM_SHARED((N_AGG, D), jnp.float32),
            pltpu.SemaphoreType.DMA,
            [pltpu.SemaphoreType.DMA for _ in range(2)],
            [pltpu.SemaphoreType.DMA for _ in range(2)],
            [pltpu.SemaphoreType.DMA for _ in range(2)],
        ],
    )


def kernel(x, edge_index, edge_attr, params, w):
    src = edge_index[0].astype(jnp.int32)
    dst = edge_index[1].astype(jnp.int32)
    pad = E_PAD - N_EDGES
    def chunked(v):
        return (v.reshape(2, TOT_CHUNKS, R)
                 .transpose(1, 0, 2)
                 .reshape(TOT_CHUNKS, CHUNK))

    src_p = chunked(jnp.concatenate([src, jnp.zeros((pad,), jnp.int32)]))
    dst_p = chunked(jnp.concatenate([dst, jnp.full((pad,), N_NODES,
                                                   jnp.int32)]))
    ea_p = jnp.concatenate(
        [edge_attr, jnp.zeros((pad, D_EDGE), jnp.float32)], axis=0)

    def permed(p):
        return _permute_cols(p["Wn"]), _permute_bias(p["bn"])

    def emm(p):
        return _emm(ea_p, _permute_cols(p["We"]), _permute_bias(p["be"]))

    e0, e1, e2, e3, e5 = (emm(params[i]) for i in (0, 1, 2, 3, 5))

    def agg_halves(hq, e):
        aggs = _sc_edge_kernel()(hq, e, src_p, dst_p)
        return aggs[0, :N_NODES, :], aggs[1, :N_NODES, :]

    skip = jnp.zeros_like(x)
    wnq0, bnq0 = permed(params[0])
    h, hq = _mm2(x, params[0]["Wn"], params[0]["bn"], wnq0, bnq0)

    seq = [
        (params[1], e0, w[0], False),
        (params[2], e1, w[1], False),
        (params[3], e2, w[2], False),
        (params[3], e3, w[3], False),
        (params[5], e3, w[4], True),
    ]
    for p_next, e_cur, wi, from_skip in seq:
        a0, a1 = agg_halves(hq, e_cur)
        wnq, bnq = permed(p_next)
        skip, h, hq = _combine_mm(h, a0, a1, skip, wi,
                                  p_next["Wn"], p_next["bn"], wnq, bnq,
                                  from_skip)

    a0, a1 = agg_halves(hq, e5)
    return _final_combine(h, a0, a1)


# speedup vs baseline: 1.0017x; 1.0017x over previous
import functools

import jax
import jax.numpy as jnp
import numpy as np
from jax import lax
from jax.experimental import pallas as pl
from jax.experimental.pallas import tpu as pltpu
from jax.experimental.pallas import tpu_sc as plsc

N_NODES = 10000
N_EDGES = 320000
D = 128
D_EDGE = 16

NC = 2
NS = 16
NW = NC * NS

CHUNK = 80
R = CHUNK // 2
E_PAD = 322560
E_HALF = E_PAD // 2
EPW = E_PAD // NW
TOT_CHUNKS = E_PAD // CHUNK
N0 = 138
N1 = 114
IDXQ = 6
N_AGG = 10112
ROWS_PER_TILE = N_AGG // NS


def _permute_cols(w):
    i = w.shape[0]
    return w.reshape(i, D // 32, 2, 16).transpose(0, 2, 1, 3).reshape(i, D)


def _permute_bias(b):
    return b.reshape(D // 32, 2, 16).transpose(1, 0, 2).reshape(D)


def _pack_bf16(v):
    lo = jax.lax.shift_right_logical(
        jax.lax.bitcast_convert_type(v[:, :D // 2], jnp.int32) + 0x8000, 16)
    hi = (jax.lax.bitcast_convert_type(v[:, D // 2:], jnp.int32)
          + 0x8000) & (-65536)
    return lo | hi


def _mm2_body(x_ref, w_ref, b_ref, wq_ref, bq_ref, o_ref, oq_ref):
    x = x_ref[...]
    o_ref[...] = (
        jnp.dot(x, w_ref[...], preferred_element_type=jnp.float32)
        + b_ref[...]
    )
    oq_ref[...] = _pack_bf16(
        jnp.dot(x, wq_ref[...], preferred_element_type=jnp.float32)
        + bq_ref[...]
    )


def _mm2(x, w, b, wq, bq, block_rows=2000):
    m, k = x.shape
    n = w.shape[1]
    grid = (m // block_rows,)
    return pl.pallas_call(
        _mm2_body,
        grid=grid,
        in_specs=[
            pl.BlockSpec((block_rows, k), lambda i: (i, 0)),
            pl.BlockSpec((k, n), lambda i: (0, 0)),
            pl.BlockSpec((1, n), lambda i: (0, 0)),
            pl.BlockSpec((k, n), lambda i: (0, 0)),
            pl.BlockSpec((1, n), lambda i: (0, 0)),
        ],
        out_specs=[
            pl.BlockSpec((block_rows, n), lambda i: (i, 0)),
            pl.BlockSpec((block_rows, n // 2), lambda i: (i, 0)),
        ],
        out_shape=[
            jax.ShapeDtypeStruct((m, n), jnp.float32),
            jax.ShapeDtypeStruct((m, n // 2), jnp.int32),
        ],
    )(x, w, b.reshape(1, n), wq, bq.reshape(1, n))


def _emm_body(xa_ref, xb_ref, w_ref, b_ref, o_ref):
    w_, b_ = w_ref[...].astype(jnp.bfloat16), b_ref[...]
    o_ref[:, :D // 2] = _pack_bf16(
        jnp.dot(xa_ref[...].astype(jnp.bfloat16), w_,
                preferred_element_type=jnp.float32) + b_)
    o_ref[:, D // 2:] = _pack_bf16(
        jnp.dot(xb_ref[...].astype(jnp.bfloat16), w_,
                preferred_element_type=jnp.float32) + b_)


def _emm(x, w, b, block_rows=1280):
    m, k = x.shape
    n = w.shape[1]
    half_blocks = (m // 2) // block_rows
    return pl.pallas_call(
        _emm_body,
        grid=(half_blocks,),
        in_specs=[
            pl.BlockSpec((block_rows, k), lambda i: (i, 0)),
            pl.BlockSpec((block_rows, k),
                         lambda i, hb=half_blocks: (i + hb, 0)),
            pl.BlockSpec((k, n), lambda i: (0, 0)),
            pl.BlockSpec((1, n), lambda i: (0, 0)),
        ],
        out_specs=pl.BlockSpec((block_rows, n), lambda i: (i, 0)),
        out_shape=jax.ShapeDtypeStruct((m // 2, n), jnp.int32),
    )(x, x, w, b.reshape(1, n))


def _combine_mm_body(h_ref, a0_ref, a1_ref, skip_ref, wi_ref, wn_ref, bn_ref,
                     wnq_ref, bnq_ref, skip_o_ref, h_o_ref, hq_o_ref,
                     *, mm_from_skip):
    xl = jnp.maximum(h_ref[...] + a0_ref[...] + a1_ref[...], 0.0)
    skip_new = skip_ref[...] + wi_ref[0, 0] * xl
    skip_o_ref[...] = skip_new
    mm_in = skip_new if mm_from_skip else xl
    h_o_ref[...] = (
        jnp.dot(mm_in, wn_ref[...], preferred_element_type=jnp.float32)
        + bn_ref[...]
    )
    hq_o_ref[...] = _pack_bf16(
        jnp.dot(mm_in, wnq_ref[...], preferred_element_type=jnp.float32)
        + bnq_ref[...]
    )


def _combine_mm(h, a0, a1, skip, wi, wn, bn, wnq, bnq, mm_from_skip,
                block_rows=2000):
    m, n = h.shape
    grid = (m // block_rows,)
    body = functools.partial(_combine_mm_body, mm_from_skip=mm_from_skip)
    bspec = pl.BlockSpec((block_rows, n), lambda i: (i, 0))
    wspec = pl.BlockSpec((n, n), lambda i: (0, 0))
    b1spec = pl.BlockSpec((1, n), lambda i: (0, 0))
    return pl.pallas_call(
        body,
        grid=grid,
        in_specs=[bspec, bspec, bspec, bspec,
                  pl.BlockSpec(memory_space=pltpu.SMEM),
                  wspec, b1spec, wspec, b1spec],
        out_specs=[bspec, bspec,
                   pl.BlockSpec((block_rows, n // 2), lambda i: (i, 0))],
        out_shape=[
            jax.ShapeDtypeStruct((m, n), jnp.float32),
            jax.ShapeDtypeStruct((m, n), jnp.float32),
            jax.ShapeDtypeStruct((m, n // 2), jnp.int32),
        ],
    )(h, a0, a1, skip, wi.reshape(1, 1), wn, bn.reshape(1, n),
      wnq, bnq.reshape(1, n))


def _final_body(h_ref, a0_ref, a1_ref, o_ref):
    o_ref[...] = jnp.maximum(h_ref[...] + a0_ref[...] + a1_ref[...], 0.0)


def _final_combine(h, a0, a1, block_rows=2000):
    m, n = h.shape
    grid = (m // block_rows,)
    spec = pl.BlockSpec((block_rows, n), lambda i: (i, 0))
    return pl.pallas_call(
        _final_body,
        grid=grid,
        in_specs=[spec, spec, spec],
        out_specs=spec,
        out_shape=jax.ShapeDtypeStruct((m, n), jnp.float32),
    )(h, a0, a1)


def _sc_edge_body(h_hbm, e_hbm, src_hbm, dst_hbm, out_hbm,
                  src_v, dst_v, g_v, e_v, m_v, agg_sh,
                  sem_i, sem_g, sem_e, sem_s):
    cid = lax.axis_index("c")
    sid = lax.axis_index("s")
    base = jnp.where(cid == 0, sid * N0, NS * N0 + sid * N1)
    nck = jnp.where(cid == 0, N0, N1)

    def issue_idx(c):
        q = c % IDXQ
        pltpu.async_copy(src_hbm.at[base + c, :], src_v.at[q], sem_i)
        pltpu.async_copy(dst_hbm.at[base + c, :], dst_v.at[q], sem_i)

    def wait_idx(c):
        q = c % IDXQ
        pltpu.make_async_copy(src_hbm.at[base + c, :], src_v.at[q],
                              sem_i).wait()
        pltpu.make_async_copy(dst_hbm.at[base + c, :], dst_v.at[q],
                              sem_i).wait()

    def issue(c, b):
        pltpu.async_copy(h_hbm.at[src_v.at[c % IDXQ]], g_v[b], sem_g[b])
        pltpu.async_copy(e_hbm.at[pl.ds((base + c) * R, R), :],
                         e_v[b], sem_e[b])

    def wait_in(c, b):
        pltpu.make_async_copy(h_hbm.at[src_v.at[c % IDXQ]], g_v[b],
                              sem_g[b]).wait()
        pltpu.make_async_copy(e_hbm.at[pl.ds((base + c) * R, R), :],
                              e_v[b], sem_e[b]).wait()

    def compute(b):
        @plsc.parallel_loop(0, R, 1, unroll=8)
        def row_body(r):
            for half in range(2):
                mr = r if half == 0 else R + r
                for j in range(D // 32):
                    gi = g_v[b][mr, pl.ds(j * 16, 16)]
                    ei = e_v[b][r, pl.ds(half * 64 + j * 16, 16)]
                    glo = lax.bitcast_convert_type(gi << 16, jnp.float32)
                    elo = lax.bitcast_convert_type(ei << 16, jnp.float32)
                    ghi = lax.bitcast_convert_type(gi & (-65536), jnp.float32)
                    ehi = lax.bitcast_convert_type(ei & (-65536), jnp.float32)
                    m_v[b][mr, pl.ds(j * 32, 16)] = jnp.maximum(glo + elo, 0.0)
                    m_v[b][mr, pl.ds(j * 32 + 16, 16)] = (
                        jnp.maximum(ghi + ehi, 0.0))

    def issue_scatter(c, b):
        pltpu.async_copy(m_v[b], agg_sh.at[dst_v.at[c % IDXQ]],
                         sem_s[b], add=True)

    def wait_scatter(c, b):
        pltpu.make_async_copy(m_v[b], agg_sh.at[dst_v.at[c % IDXQ]],
                              sem_s[b]).wait()

    for c in range(4):
        issue_idx(c)

    def zero_row(r, _):
        for j in range(D // 16):
            m_v[0][r, pl.ds(j * 16, 16)] = jnp.zeros((16,), jnp.float32)
        return 0
    lax.fori_loop(0, CHUNK, zero_row, 0)
    row0 = sid * ROWS_PER_TILE
    for k in range((ROWS_PER_TILE + CHUNK - 1) // CHUNK):
        rows = min(CHUNK, ROWS_PER_TILE - k * CHUNK)
        pltpu.sync_copy(m_v[0].at[pl.ds(0, rows), :],
                        agg_sh.at[pl.ds(row0 + k * CHUNK, rows), :])
    plsc.subcore_barrier()

    wait_idx(0)
    wait_idx(1)
    for b in range(2):
        issue(b, b)
    for c in range(2):
        b = c
        wait_in(c, b)
        compute(b)
        wait_idx(c + 2)
        issue(c + 2, b)
        issue_idx(c + 4)
        issue_scatter(c, b)

    def pair_body(i, _):
        for b in range(2):
            c = 2 * i + b
            wait_in(c, b)
            wait_scatter(c - 2, b)
            compute(b)
            wait_idx(c + 2)
            issue(c + 2, b)
            issue_idx(c + 4)
            issue_scatter(c, b)
        return 0
    lax.fori_loop(1, nck // 2 - 2, pair_body, 0)

    for k in range(4):
        c = nck - 4 + k
        b = k % 2
        wait_in(c, b)
        wait_scatter(c - 2, b)
        compute(b)
        if k < 2:
            wait_idx(c + 2)
            issue(c + 2, b)
        issue_scatter(c, b)
    for b in range(2):
        wait_scatter(nck - 2 + b, b)
    plsc.subcore_barrier()

    for k in range((ROWS_PER_TILE + CHUNK - 1) // CHUNK):
        rows = min(CHUNK, ROWS_PER_TILE - k * CHUNK)
        pltpu.sync_copy(agg_sh.at[pl.ds(row0 + k * CHUNK, rows), :],
                        m_v[0].at[pl.ds(0, rows), :])
        pltpu.sync_copy(m_v[0].at[pl.ds(0, rows), :],
                        out_hbm.at[cid, pl.ds(row0 + k * CHUNK, rows), :])


@functools.lru_cache(maxsize=None)
def _sc_edge_kernel():
    return pl.kernel(
        _sc_edge_body,
        out_type=jax.ShapeDtypeStruct((NC, N_AGG, D), jnp.float32),
        mesh=plsc.VectorSubcoreMesh(core_axis_name="c", subcore_axis_name="s",
                                    num_cores=NC, num_subcores=NS),
        compiler_params=pltpu.CompilerParams(use_tc_tiling_on_sc=False),
        scratch_types=[
            pltpu.VMEM((IDXQ, CHUNK), jnp.int32),
            pltpu.VMEM((IDXQ, CHUNK), jnp.int32),
            [pltpu.VMEM((CHUNK, D // 2), jnp.int32) for _ in range(2)],
            [pltpu.VMEM((R, D), jnp.int32) for _ in range(2)],
            [pltpu.VMEM((CHUNK, D), jnp.float32) for _ in range(2)],
            pltpu.VMEM_SHARED((N_AGG, D), jnp.float32),
            pltpu.SemaphoreType.DMA,
            [pltpu.SemaphoreType.DMA for _ in range(2)],
            [pltpu.SemaphoreType.DMA for _ in range(2)],
            [pltpu.SemaphoreType.DMA for _ in range(2)],
        ],
    )


def kernel(x, edge_index, edge_attr, params, w):
    src = edge_index[0].astype(jnp.int32)
    dst = edge_index[1].astype(jnp.int32)
    pad = E_PAD - N_EDGES
    def chunked(v):
        return (v.reshape(2, TOT_CHUNKS, R)
                 .transpose(1, 0, 2)
                 .reshape(TOT_CHUNKS, CHUNK))

    src_p = chunked(jnp.concatenate([src, jnp.zeros((pad,), jnp.int32)]))
    dst_p = chunked(jnp.concatenate([dst, jnp.full((pad,), N_NODES,
                                                   jnp.int32)]))
    ea_p = jnp.concatenate(
        [edge_attr, jnp.zeros((pad, D_EDGE), jnp.float32)], axis=0)

    def permed(p):
        return _permute_cols(p["Wn"]), _permute_bias(p["bn"])

    def emm(p):
        return _emm(ea_p, _permute_cols(p["We"]), _permute_bias(p["be"]))

    e0, e1, e2, e3, e5 = (emm(params[i]) for i in (0, 1, 2, 3, 5))

    def agg_halves(hq, e):
        aggs = _sc_edge_kernel()(hq, e, src_p, dst_p)
        return aggs[0, :N_NODES, :], aggs[1, :N_NODES, :]

    skip = jnp.zeros_like(x)
    wnq0, bnq0 = permed(params[0])
    h, hq = _mm2(x, params[0]["Wn"], params[0]["bn"], wnq0, bnq0)

    seq = [
        (params[1], e0, w[0], False),
        (params[2], e1, w[1], False),
        (params[3], e2, w[2], False),
        (params[3], e3, w[3], False),
        (params[5], e3, w[4], True),
    ]
    for p_next, e_cur, wi, from_skip in seq:
        a0, a1 = agg_halves(hq, e_cur)
        wnq, bnq = permed(p_next)
        skip, h, hq = _combine_mm(h, a0, a1, skip, wi,
                                  p_next["Wn"], p_next["bn"], wnq, bnq,
                                  from_skip)

    a0, a1 = agg_halves(hq, e5)
    return _final_combine(h, a0, a1)


# speedup vs baseline: 1.0182x; 1.0165x over previous
import functools

import jax
import jax.numpy as jnp
import numpy as np
from jax import lax
from jax.experimental import pallas as pl
from jax.experimental.pallas import tpu as pltpu
from jax.experimental.pallas import tpu_sc as plsc

N_NODES = 10000
N_EDGES = 320000
D = 128
D_EDGE = 16

NC = 2
NS = 16
NW = NC * NS

CHUNK = 80
R = CHUNK // 2
E_PAD = 322560
E_HALF = E_PAD // 2
EPW = E_PAD // NW
TOT_CHUNKS = E_PAD // CHUNK
N0 = 138
N1 = 114
IDXQ = 6
N_AGG = 10112
ROWS_PER_TILE = N_AGG // NS


def _permute_cols(w):
    i = w.shape[0]
    return w.reshape(i, D // 32, 2, 16).transpose(0, 2, 1, 3).reshape(i, D)


def _permute_bias(b):
    return b.reshape(D // 32, 2, 16).transpose(1, 0, 2).reshape(D)


def _pack_bf16(v):
    lo = jax.lax.shift_right_logical(
        jax.lax.bitcast_convert_type(v[:, :D // 2], jnp.int32) + 0x8000, 16)
    hi = (jax.lax.bitcast_convert_type(v[:, D // 2:], jnp.int32)
          + 0x8000) & (-65536)
    return lo | hi


def _mm2_body(x_ref, w_ref, b_ref, wq_ref, bq_ref, o_ref, oq_ref):
    x = x_ref[...]
    o_ref[...] = (
        jnp.dot(x, w_ref[...], preferred_element_type=jnp.float32)
        + b_ref[...]
    )
    oq_ref[...] = _pack_bf16(
        jnp.dot(x, wq_ref[...], preferred_element_type=jnp.float32)
        + bq_ref[...]
    )


def _mm2(x, w, b, wq, bq, block_rows=2000):
    m, k = x.shape
    n = w.shape[1]
    grid = (m // block_rows,)
    return pl.pallas_call(
        _mm2_body,
        grid=grid,
        in_specs=[
            pl.BlockSpec((block_rows, k), lambda i: (i, 0)),
            pl.BlockSpec((k, n), lambda i: (0, 0)),
            pl.BlockSpec((1, n), lambda i: (0, 0)),
            pl.BlockSpec((k, n), lambda i: (0, 0)),
            pl.BlockSpec((1, n), lambda i: (0, 0)),
        ],
        out_specs=[
            pl.BlockSpec((block_rows, n), lambda i: (i, 0)),
            pl.BlockSpec((block_rows, n // 2), lambda i: (i, 0)),
        ],
        out_shape=[
            jax.ShapeDtypeStruct((m, n), jnp.float32),
            jax.ShapeDtypeStruct((m, n // 2), jnp.int32),
        ],
    )(x, w, b.reshape(1, n), wq, bq.reshape(1, n))


def _emm_body(xa_ref, xb_ref, w_ref, b_ref, o_ref):
    w_, b_ = w_ref[...].astype(jnp.bfloat16), b_ref[...]
    o_ref[:, :D // 2] = _pack_bf16(
        jnp.dot(xa_ref[...].astype(jnp.bfloat16), w_,
                preferred_element_type=jnp.float32) + b_)
    o_ref[:, D // 2:] = _pack_bf16(
        jnp.dot(xb_ref[...].astype(jnp.bfloat16), w_,
                preferred_element_type=jnp.float32) + b_)


def _emm(x, w, b, block_rows=1280):
    m, k = x.shape
    n = w.shape[1]
    half_blocks = (m // 2) // block_rows
    return pl.pallas_call(
        _emm_body,
        grid=(half_blocks,),
        in_specs=[
            pl.BlockSpec((block_rows, k), lambda i: (i, 0)),
            pl.BlockSpec((block_rows, k),
                         lambda i, hb=half_blocks: (i + hb, 0)),
            pl.BlockSpec((k, n), lambda i: (0, 0)),
            pl.BlockSpec((1, n), lambda i: (0, 0)),
        ],
        out_specs=pl.BlockSpec((block_rows, n), lambda i: (i, 0)),
        out_shape=jax.ShapeDtypeStruct((m // 2, n), jnp.int32),
    )(x, x, w, b.reshape(1, n))


def _combine_mm_body(h_ref, a0_ref, a1_ref, skip_ref, wi_ref, wn_ref, bn_ref,
                     wnq_ref, bnq_ref, skip_o_ref, h_o_ref, hq_o_ref,
                     *, mm_from_skip):
    xl = jnp.maximum(h_ref[...] + a0_ref[0] + a1_ref[0], 0.0)
    skip_new = skip_ref[...] + wi_ref[0, 0] * xl
    skip_o_ref[...] = skip_new
    mm_in = skip_new if mm_from_skip else xl
    h_o_ref[...] = (
        jnp.dot(mm_in, wn_ref[...], preferred_element_type=jnp.float32)
        + bn_ref[...]
    )
    hq_o_ref[...] = _pack_bf16(
        jnp.dot(mm_in, wnq_ref[...], preferred_element_type=jnp.float32)
        + bnq_ref[...]
    )


def _combine_mm(h, aggs, skip, wi, wn, bn, wnq, bnq, mm_from_skip,
                block_rows=2000):
    m, n = h.shape
    grid = (m // block_rows,)
    body = functools.partial(_combine_mm_body, mm_from_skip=mm_from_skip)
    bspec = pl.BlockSpec((block_rows, n), lambda i: (i, 0))
    wspec = pl.BlockSpec((n, n), lambda i: (0, 0))
    b1spec = pl.BlockSpec((1, n), lambda i: (0, 0))
    aspec0 = pl.BlockSpec((1, block_rows, n), lambda i: (0, i, 0))
    aspec1 = pl.BlockSpec((1, block_rows, n), lambda i: (1, i, 0))
    return pl.pallas_call(
        body,
        grid=grid,
        in_specs=[bspec, aspec0, aspec1, bspec,
                  pl.BlockSpec(memory_space=pltpu.SMEM),
                  wspec, b1spec, wspec, b1spec],
        out_specs=[bspec, bspec,
                   pl.BlockSpec((block_rows, n // 2), lambda i: (i, 0))],
        out_shape=[
            jax.ShapeDtypeStruct((m, n), jnp.float32),
            jax.ShapeDtypeStruct((m, n), jnp.float32),
            jax.ShapeDtypeStruct((m, n // 2), jnp.int32),
        ],
    )(h, aggs, aggs, skip, wi.reshape(1, 1), wn, bn.reshape(1, n),
      wnq, bnq.reshape(1, n))


def _final_body(h_ref, a0_ref, a1_ref, o_ref):
    o_ref[...] = jnp.maximum(h_ref[...] + a0_ref[0] + a1_ref[0], 0.0)


def _final_combine(h, aggs, block_rows=2000):
    m, n = h.shape
    grid = (m // block_rows,)
    spec = pl.BlockSpec((block_rows, n), lambda i: (i, 0))
    return pl.pallas_call(
        _final_body,
        grid=grid,
        in_specs=[spec,
                  pl.BlockSpec((1, block_rows, n), lambda i: (0, i, 0)),
                  pl.BlockSpec((1, block_rows, n), lambda i: (1, i, 0))],
        out_specs=spec,
        out_shape=jax.ShapeDtypeStruct((m, n), jnp.float32),
    )(h, aggs, aggs)


def _sc_edge_body(h_hbm, e_hbm, src_hbm, dst_hbm, out_hbm,
                  src_v, dst_v, g_v, e_v, m_v, agg_sh,
                  sem_i, sem_g, sem_e, sem_s):
    cid = lax.axis_index("c")
    sid = lax.axis_index("s")
    base = jnp.where(cid == 0, sid * N0, NS * N0 + sid * N1)
    nck = jnp.where(cid == 0, N0, N1)

    def issue_idx(c):
        q = c % IDXQ
        pltpu.async_copy(src_hbm.at[base + c, :], src_v.at[q], sem_i)
        pltpu.async_copy(dst_hbm.at[base + c, :], dst_v.at[q], sem_i)

    def wait_idx(c):
        q = c % IDXQ
        pltpu.make_async_copy(src_hbm.at[base + c, :], src_v.at[q],
                              sem_i).wait()
        pltpu.make_async_copy(dst_hbm.at[base + c, :], dst_v.at[q],
                              sem_i).wait()

    def issue(c, b):
        pltpu.async_copy(h_hbm.at[src_v.at[c % IDXQ]], g_v[b], sem_g[b])
        pltpu.async_copy(e_hbm.at[pl.ds((base + c) * R, R), :],
                         e_v[b], sem_e[b])

    def wait_in(c, b):
        pltpu.make_async_copy(h_hbm.at[src_v.at[c % IDXQ]], g_v[b],
                              sem_g[b]).wait()
        pltpu.make_async_copy(e_hbm.at[pl.ds((base + c) * R, R), :],
                              e_v[b], sem_e[b]).wait()

    def compute(b):
        @plsc.parallel_loop(0, R, 1, unroll=8)
        def row_body(r):
            for half in range(2):
                mr = r if half == 0 else R + r
                for j in range(D // 32):
                    gi = g_v[b][mr, pl.ds(j * 16, 16)]
                    ei = e_v[b][r, pl.ds(half * 64 + j * 16, 16)]
                    glo = lax.bitcast_convert_type(gi << 16, jnp.float32)
                    elo = lax.bitcast_convert_type(ei << 16, jnp.float32)
                    ghi = lax.bitcast_convert_type(gi & (-65536), jnp.float32)
                    ehi = lax.bitcast_convert_type(ei & (-65536), jnp.float32)
                    m_v[b][mr, pl.ds(j * 32, 16)] = jnp.maximum(glo + elo, 0.0)
                    m_v[b][mr, pl.ds(j * 32 + 16, 16)] = (
                        jnp.maximum(ghi + ehi, 0.0))

    def issue_scatter(c, b):
        pltpu.async_copy(m_v[b], agg_sh.at[dst_v.at[c % IDXQ]],
                         sem_s[b], add=True)

    def wait_scatter(c, b):
        pltpu.make_async_copy(m_v[b], agg_sh.at[dst_v.at[c % IDXQ]],
                              sem_s[b]).wait()

    for c in range(4):
        issue_idx(c)

    def zero_row(r, _):
        for j in range(D // 16):
            m_v[0][r, pl.ds(j * 16, 16)] = jnp.zeros((16,), jnp.float32)
        return 0
    lax.fori_loop(0, CHUNK, zero_row, 0)
    row0 = sid * ROWS_PER_TILE
    for k in range((ROWS_PER_TILE + CHUNK - 1) // CHUNK):
        rows = min(CHUNK, ROWS_PER_TILE - k * CHUNK)
        pltpu.sync_copy(m_v[0].at[pl.ds(0, rows), :],
                        agg_sh.at[pl.ds(row0 + k * CHUNK, rows), :])
    plsc.subcore_barrier()

    wait_idx(0)
    wait_idx(1)
    for b in range(2):
        issue(b, b)
    for c in range(2):
        b = c
        wait_in(c, b)
        compute(b)
        wait_idx(c + 2)
        issue(c + 2, b)
        issue_idx(c + 4)
        issue_scatter(c, b)

    def pair_body(i, _):
        for b in range(2):
            c = 2 * i + b
            wait_in(c, b)
            wait_scatter(c - 2, b)
            compute(b)
            wait_idx(c + 2)
            issue(c + 2, b)
            issue_idx(c + 4)
            issue_scatter(c, b)
        return 0
    lax.fori_loop(1, nck // 2 - 2, pair_body, 0)

    for k in range(4):
        c = nck - 4 + k
        b = k % 2
        wait_in(c, b)
        wait_scatter(c - 2, b)
        compute(b)
        if k < 2:
            wait_idx(c + 2)
            issue(c + 2, b)
        issue_scatter(c, b)
    for b in range(2):
        wait_scatter(nck - 2 + b, b)
    plsc.subcore_barrier()

    for k in range((ROWS_PER_TILE + CHUNK - 1) // CHUNK):
        rows = min(CHUNK, ROWS_PER_TILE - k * CHUNK)
        pltpu.sync_copy(agg_sh.at[pl.ds(row0 + k * CHUNK, rows), :],
                        m_v[0].at[pl.ds(0, rows), :])
        pltpu.sync_copy(m_v[0].at[pl.ds(0, rows), :],
                        out_hbm.at[cid, pl.ds(row0 + k * CHUNK, rows), :])


@functools.lru_cache(maxsize=None)
def _sc_edge_kernel():
    return pl.kernel(
        _sc_edge_body,
        out_type=jax.ShapeDtypeStruct((NC, N_AGG, D), jnp.float32),
        mesh=plsc.VectorSubcoreMesh(core_axis_name="c", subcore_axis_name="s",
                                    num_cores=NC, num_subcores=NS),
        compiler_params=pltpu.CompilerParams(use_tc_tiling_on_sc=False),
        scratch_types=[
            pltpu.VMEM((IDXQ, CHUNK), jnp.int32),
            pltpu.VMEM((IDXQ, CHUNK), jnp.int32),
            [pltpu.VMEM((CHUNK, D // 2), jnp.int32) for _ in range(2)],
            [pltpu.VMEM((R, D), jnp.int32) for _ in range(2)],
            [pltpu.VMEM((CHUNK, D), jnp.float32) for _ in range(2)],
            pltpu.VMEM_SHARED((N_AGG, D), jnp.float32),
            pltpu.SemaphoreType.DMA,
            [pltpu.SemaphoreType.DMA for _ in range(2)],
            [pltpu.SemaphoreType.DMA for _ in range(2)],
            [pltpu.SemaphoreType.DMA for _ in range(2)],
        ],
    )


def kernel(x, edge_index, edge_attr, params, w):
    src = edge_index[0].astype(jnp.int32)
    dst = edge_index[1].astype(jnp.int32)
    pad = E_PAD - N_EDGES
    def chunked(v):
        return (v.reshape(2, TOT_CHUNKS, R)
                 .transpose(1, 0, 2)
                 .reshape(TOT_CHUNKS, CHUNK))

    src_p = chunked(jnp.concatenate([src, jnp.zeros((pad,), jnp.int32)]))
    dst_p = chunked(jnp.concatenate([dst, jnp.full((pad,), N_NODES,
                                                   jnp.int32)]))
    ea_p = jnp.concatenate(
        [edge_attr, jnp.zeros((pad, D_EDGE), jnp.float32)], axis=0)

    def permed(p):
        return _permute_cols(p["Wn"]), _permute_bias(p["bn"])

    def emm(p):
        return _emm(ea_p, _permute_cols(p["We"]), _permute_bias(p["be"]))

    e0, e1, e2, e3, e5 = (emm(params[i]) for i in (0, 1, 2, 3, 5))

    def sc_pass(hq, e):
        return _sc_edge_kernel()(hq, e, src_p, dst_p)

    skip = jnp.zeros_like(x)
    wnq0, bnq0 = permed(params[0])
    h, hq = _mm2(x, params[0]["Wn"], params[0]["bn"], wnq0, bnq0)

    seq = [
        (params[1], e0, w[0], False),
        (params[2], e1, w[1], False),
        (params[3], e2, w[2], False),
        (params[3], e3, w[3], False),
        (params[5], e3, w[4], True),
    ]
    for p_next, e_cur, wi, from_skip in seq:
        aggs = sc_pass(hq, e_cur)
        wnq, bnq = permed(p_next)
        skip, h, hq = _combine_mm(h, aggs, skip, wi,
                                  p_next["Wn"], p_next["bn"], wnq, bnq,
                                  from_skip)

    aggs = sc_pass(hq, e5)
    return _final_combine(h, aggs)
